# Initial kernel scaffold; baseline (speedup 1.0000x reference)
import jax
import jax.numpy as jnp
from jax.experimental import pallas as pl


def kernel(embedded_node, embedded_adjancy_matrix, Wf, bf, Wih, Whh, bih, bhh):
    raise NotImplementedError("write your pallas kernel here")


# speedup vs baseline: 1.4936x; 1.4936x over previous
import jax
import jax.numpy as jnp
from jax.experimental import pallas as pl
from jax.experimental.pallas import tpu as pltpu

_F32 = jnp.float32
_BF16 = jnp.bfloat16


def _dot(a, b, precision=None):
    return jax.lax.dot_general(
        a, b, (((1,), (0,)), ((), ())),
        precision=precision,
        preferred_element_type=_F32,
    )


def _prop_kernel(e_ref, node_ref, wfab_ref, wfc2_ref, bf_ref, wih_ref,
                 whh_ref, bih_ref, bhh_ref, out_ref):
    n = node_ref.shape[1]
    h_dim = node_ref.shape[2]
    r_rounds = wfab_ref.shape[0]
    hi = jax.lax.Precision.HIGHEST

    y = e_ref[0, :, 0, :].astype(_BF16).astype(_F32)
    for q in range(1, e_ref.shape[2]):
        y = y + e_ref[0, :, q, :].astype(_BF16).astype(_F32)

    h = node_ref[0]
    fn = _F32(n)
    for t in range(r_rounds):
        h16 = h.astype(_BF16)
        hterm = _dot(h16, wfab_ref[t, :h_dim, :]) * fn
        hs = jnp.sum(h16.astype(_F32), axis=0, keepdims=True)
        hsterm = _dot(hs, wfab_ref[t, h_dim:, :].astype(_F32), hi)
        eterm = _dot(y, wfc2_ref[t].astype(_F32), hi)
        agg = (hterm + jnp.broadcast_to(hsterm, (n, 6 * h_dim)) + eterm
               + fn * bf_ref[t][None, :])
        gi = _dot(agg.astype(_BF16), wih_ref[t]) + bih_ref[t][None, :]
        gh = _dot(h16, whh_ref[t]) + bhh_ref[t][None, :]
        i_r, i_z, i_n = jnp.split(gi, 3, axis=-1)
        h_r, h_z, h_n = jnp.split(gh, 3, axis=-1)
        r = jax.nn.sigmoid(i_r + h_r)
        z = jax.nn.sigmoid(i_z + h_z)
        nn = jnp.tanh(i_n + r * h_n)
        h = (1.0 - z) * nn + z * h

    out_ref[...] = jnp.sum(h, axis=0, keepdims=True)[None]


def kernel(embedded_node, embedded_adjancy_matrix, Wf, bf, Wih, Whh, bih, bhh):
    b_g, n, _, h_dim = embedded_adjancy_matrix.shape
    r_rounds = Wf.shape[0]
    e_r = embedded_adjancy_matrix.reshape(b_g, n, h_dim, 2 * h_dim)
    wfab16 = Wf[:, :2 * h_dim, :].astype(_BF16)
    wfc2_16 = jnp.concatenate(
        [Wf[:, 2 * h_dim:, :], Wf[:, 2 * h_dim:, :]], axis=1).astype(_BF16)
    wih16 = Wih.astype(_BF16)
    whh16 = Whh.astype(_BF16)

    grid = (b_g,)
    out = pl.pallas_call(
        _prop_kernel,
        grid=grid,
        in_specs=[
            pl.BlockSpec((1, n, h_dim, 2 * h_dim), lambda b: (b, 0, 0, 0)),
            pl.BlockSpec((1, n, h_dim), lambda b: (b, 0, 0)),
            pl.BlockSpec((r_rounds, 2 * h_dim, 6 * h_dim), lambda b: (0, 0, 0)),
            pl.BlockSpec((r_rounds, 2 * h_dim, 6 * h_dim), lambda b: (0, 0, 0)),
            pl.BlockSpec((r_rounds, 6 * h_dim), lambda b: (0, 0)),
            pl.BlockSpec((r_rounds, 6 * h_dim, 3 * h_dim), lambda b: (0, 0, 0)),
            pl.BlockSpec((r_rounds, h_dim, 3 * h_dim), lambda b: (0, 0, 0)),
            pl.BlockSpec((r_rounds, 3 * h_dim), lambda b: (0, 0)),
            pl.BlockSpec((r_rounds, 3 * h_dim), lambda b: (0, 0)),
        ],
        out_specs=pl.BlockSpec((1, 1, h_dim), lambda b: (b, 0, 0)),
        out_shape=jax.ShapeDtypeStruct((b_g, 1, h_dim), jnp.float32),
        compiler_params=pltpu.CompilerParams(
            dimension_semantics=("arbitrary",),
        ),
    )(e_r, embedded_node, wfab16, wfc2_16, bf, wih16, whh16, bih, bhh)
    return out.reshape(b_g, h_dim)
